# baseline (speedup 1.0000x reference)
import jax
import jax.numpy as jnp
from jax import lax
from jax.experimental import pallas as pl
from jax.experimental.pallas import tpu as pltpu
from jax.experimental.pallas import tpu_sc as plsc

B = 2
N = 10000
F = 128
E = 160000

NT = 16
L = 16

N_PAD = 10240
EPT = E // NT
K = 64
NCHUNK = -(-EPT // K)
EPT_P = NCHUNK * K
NPT = N_PAD // NT
RSUB = K
NSUB = NPT // RSUB
NTAIL = N % RSUB
HR = N_PAD // F


def _body(x_hbm, idx_hbm, out_hbm,
          acc_sp, cnt_sp, idx_v, stg_v, rows_v, hist_v, idn_v,
          g0, g1, g2, s0, s1, s2, zsem):
  cid = lax.axis_index("c")
  sid = lax.axis_index("s")
  gsem = (g0, g1, g2)
  ssem = (s0, s1, s2)

  zero16 = jnp.zeros((L,), jnp.float32)
  one16 = jnp.ones((L,), jnp.float32)
  iota16 = lax.iota(jnp.int32, L)

  def rows_init(i, _):
    for p in range(3):
      for j in range(F // L):
        rows_v[p, i, pl.ds(j * L, L)] = zero16
    return 0
  lax.fori_loop(0, RSUB, rows_init, 0)

  def hist_init(i, _):
    for j in range(F // L):
      hist_v[i, pl.ds(j * L, L)] = zero16
    return 0
  lax.fori_loop(0, HR, hist_init, 0)

  for j in range(HR // L):
    idn_v[pl.ds(j * L, L)] = iota16 + j * L

  for q in range(NSUB):
    pltpu.async_copy(rows_v.at[q % 3],
                     acc_sp.at[pl.ds(sid * NPT + q * RSUB, RSUB)], zsem)

  @pl.when(sid == 0)
  def _():
    pltpu.sync_copy(rows_v.at[0], cnt_sp.at[pl.ds(0, RSUB)])
    pltpu.sync_copy(rows_v.at[1, pl.ds(0, HR - RSUB)],
                    cnt_sp.at[pl.ds(RSUB, HR - RSUB)])

  pltpu.sync_copy(idx_hbm.at[pl.ds((cid * NT + sid) * EPT_P, EPT_P)], idx_v)

  for q in range(NSUB):
    pltpu.make_async_copy(rows_v.at[0], acc_sp.at[pl.ds(0, RSUB)],
                          zsem).wait()

  plsc.subcore_barrier()

  def unpack(c, s):
    for j in range(K // L):
      w = idx_v[pl.ds(c * K + j * L, L)]
      stg_v[s, 0, pl.ds(j * L, L)] = w & 0xFFFF
      stg_v[s, 1, pl.ds(j * L, L)] = lax.shift_right_logical(w, 16)

  def hist_update(c):
    for j in range(K // L):
      w = idx_v[pl.ds(c * K + j * L, L)]
      d = lax.shift_right_logical(w, 16)
      plsc.addupdate_scatter(
          hist_v, [lax.shift_right_logical(d, 7), d & 0x7F], one16)

  def start_gather(p):
    pltpu.async_copy(x_hbm.at[stg_v.at[p, 0]], rows_v.at[p], gsem[p])

  def start_scatter(p):
    pltpu.async_copy(rows_v.at[p], acc_sp.at[stg_v.at[p, 1]], ssem[p],
                     add=True)

  def wait_gather(p):
    pltpu.make_async_copy(x_hbm.at[pl.ds(0, K)], rows_v.at[p],
                          gsem[p]).wait()

  def wait_scatter(p):
    pltpu.make_async_copy(rows_v.at[p], acc_sp.at[pl.ds(0, K)],
                          ssem[p]).wait()

  unpack(0, 0)
  start_gather(0)
  unpack(1, 1)
  start_gather(1)
  wait_gather(0)
  start_scatter(0)
  unpack(2, 2)
  start_gather(2)
  hist_update(0)

  def pipe_triple(g, _):
    for dp in range(3):
      c = 1 + 3 * g + dp
      p = (1 + dp) % 3
      wait_gather(p)
      start_scatter(p)
      wait_scatter(dp)
      cg = jnp.minimum(c + 2, NCHUNK - 1)
      unpack(cg, dp)
      start_gather(dp)
      hist_update(c)
    return 0
  lax.fori_loop(0, (NCHUNK - 1) // 3, pipe_triple, 0)

  wait_gather(1)
  wait_gather(2)
  wait_scatter(0)

  pltpu.sync_copy(hist_v, cnt_sp.at[idn_v], add=True)
  plsc.subcore_barrier()

  lo = sid * (NPT // F)
  floor8 = pl.multiple_of(jnp.minimum((lo // 8) * 8, HR - 16), 8)
  skew = lo - floor8
  pltpu.sync_copy(cnt_sp.at[pl.ds(floor8, 16)], hist_v.at[pl.ds(0, 16)])

  def cnt_inv(i, _):
    row = skew + (i // 8)
    col = (i % 8) * L
    v = hist_v[row, pl.ds(col, L)]
    hist_v[16 + i // 8, pl.ds(col, L)] = 1.0 / jnp.maximum(v, 1.0)
    return 0
  lax.fori_loop(0, NPT // L, cnt_inv, 0)

  for q in range(NSUB):
    base = sid * NPT + q * RSUB

    @pl.when(base < N)
    def _(q=q, base=base):
      pltpu.sync_copy(acc_sp.at[pl.ds(base, RSUB)], rows_v.at[0])

      def scale_grp(g, _, q=q):
        n = q * RSUB + g * L
        cvec = hist_v[16 + n // F, pl.ds(n % F, L)]
        for k in range(L):
          inv = cvec[k]
          for j in range(F // L):
            sl = pl.ds(j * L, L)
            rows_v[0, g * L + k, sl] = rows_v[0, g * L + k, sl] * inv
        return 0
      lax.fori_loop(0, RSUB // L, scale_grp, 0)

      @pl.when(base + RSUB <= N)
      def _():
        pltpu.sync_copy(rows_v.at[0], out_hbm.at[pl.ds(cid * N + base, RSUB)])

      @pl.when(base + RSUB > N)
      def _():
        pltpu.sync_copy(rows_v.at[0, pl.ds(0, NTAIL)],
                        out_hbm.at[pl.ds(cid * N + base, NTAIL)])


@jax.jit
def _graph_layer(x2, idx_all):
  mesh = plsc.VectorSubcoreMesh(core_axis_name="c", subcore_axis_name="s")
  return pl.kernel(
      _body,
      out_type=jax.ShapeDtypeStruct((B * N, F), jnp.float32),
      mesh=mesh,
      compiler_params=pltpu.CompilerParams(needs_layout_passes=False),
      scratch_types=[
          pltpu.VMEM_SHARED((N_PAD, F), jnp.float32),
          pltpu.VMEM_SHARED((HR, F), jnp.float32),
          pltpu.VMEM((EPT_P,), jnp.int32),
          pltpu.VMEM((3, 2, K), jnp.int32),
          pltpu.VMEM((3, K, F), jnp.float32),
          pltpu.VMEM((HR, F), jnp.float32),
          pltpu.VMEM((HR,), jnp.int32),
          pltpu.SemaphoreType.DMA,
          pltpu.SemaphoreType.DMA,
          pltpu.SemaphoreType.DMA,
          pltpu.SemaphoreType.DMA,
          pltpu.SemaphoreType.DMA,
          pltpu.SemaphoreType.DMA,
          pltpu.SemaphoreType.DMA,
      ],
  )(x2, idx_all)


def kernel(X, edge_index):
  x2 = X.reshape(B * N, F)
  src = edge_index[0].reshape(NT, EPT)
  dst = edge_index[1].reshape(NT, EPT)
  pads = ((0, 0), (0, EPT_P - EPT))
  packed = jnp.stack(
      [jnp.pad((src + b * N) | (dst << 16), pads,
               constant_values=(b * N) | (N << 16)) for b in range(B)])
  idx_all = packed.reshape(-1)
  out2 = _graph_layer(x2, idx_all)
  return out2.reshape(B, N, F)


# speedup vs baseline: 1.7214x; 1.7214x over previous
import jax
import jax.numpy as jnp
from jax import lax
from jax.experimental import pallas as pl
from jax.experimental.pallas import tpu as pltpu
from jax.experimental.pallas import tpu_sc as plsc

B = 2
N = 10000
F = 128
E = 160000

NT = 16
L = 16

N_PAD = 10240
EPT = E // NT
K = 80
NCHUNK = EPT // K
NPT = N_PAD // NT
RSUB = K
NSUB = NPT // RSUB
HR = N_PAD // F


def _body(x_hbm, idx_hbm, out_hbm,
          acc_sp, cnt_sp, idx_v, stg_v, rows_v, idn_v,
          g0, g1, g2, s0, s1, s2, zsem):
  cid = lax.axis_index("c")
  sid = lax.axis_index("s")
  gsem = (g0, g1, g2)
  ssem = (s0, s1, s2)

  zero16 = jnp.zeros((L,), jnp.float32)
  one16 = jnp.ones((L,), jnp.float32)
  iota16 = lax.iota(jnp.int32, L)

  def rows_init(i, _):
    for p in range(3):
      for j in range(F // L):
        rows_v[p, i, pl.ds(j * L, L)] = zero16
    return 0
  lax.fori_loop(0, RSUB, rows_init, 0)

  for j in range(HR // L):
    idn_v[pl.ds(j * L, L)] = iota16 + j * L

  for q in range(NSUB):
    pltpu.async_copy(rows_v.at[q % 3],
                     acc_sp.at[pl.ds(sid * NPT + q * RSUB, RSUB)], zsem)

  @pl.when(sid == 0)
  def _():
    pltpu.sync_copy(rows_v.at[0], cnt_sp)

  pltpu.sync_copy(idx_hbm.at[pl.ds((cid * NT + sid) * EPT, EPT)], idx_v)

  for q in range(NSUB):
    pltpu.make_async_copy(rows_v.at[0], acc_sp.at[pl.ds(0, RSUB)],
                          zsem).wait()

  plsc.subcore_barrier()

  def unpack(c, s):
    for j in range(K // L):
      w = idx_v[pl.ds(c * K + j * L, L)]
      stg_v[s, 0, pl.ds(j * L, L)] = w & 0xFFFF
      stg_v[s, 1, pl.ds(j * L, L)] = lax.shift_right_logical(w, 16)

  def start_gather(p):
    pltpu.async_copy(x_hbm.at[stg_v.at[p, 0]], rows_v.at[p], gsem[p])

  def start_scatter(p):
    pltpu.async_copy(rows_v.at[p], acc_sp.at[stg_v.at[p, 1]], ssem[p],
                     add=True)

  def wait_gather(p):
    pltpu.make_async_copy(x_hbm.at[pl.ds(0, K)], rows_v.at[p],
                          gsem[p]).wait()

  def wait_scatter(p):
    pltpu.make_async_copy(rows_v.at[p], acc_sp.at[pl.ds(0, K)],
                          ssem[p]).wait()

  unpack(0, 0)
  start_gather(0)
  unpack(1, 1)
  start_gather(1)
  wait_gather(0)
  start_scatter(0)
  unpack(2, 2)
  start_gather(2)

  def pipe_triple(g, _):
    for dp in range(3):
      c = 1 + 3 * g + dp
      p = (1 + dp) % 3
      wait_gather(p)
      start_scatter(p)
      wait_scatter(dp)
      cg = jnp.minimum(c + 2, NCHUNK - 1)
      unpack(cg, dp)
      start_gather(dp)
    return 0
  lax.fori_loop(0, (NCHUNK - 2) // 3, pipe_triple, 0)

  wait_gather(1)
  start_scatter(1)
  wait_scatter(0)
  wait_gather(2)
  wait_scatter(1)

  hist = rows_v.at[0]

  def hist_zero(i, _):
    for j in range(F // L):
      rows_v[0, i, pl.ds(j * L, L)] = zero16
    return 0
  lax.fori_loop(0, RSUB, hist_zero, 0)

  def hist_pass(e, _):
    w = idx_v[pl.ds(e * L, L)]
    d = lax.shift_right_logical(w, 16)
    plsc.addupdate_scatter(
        hist, [lax.shift_right_logical(d, 7), d & 0x7F], one16)
    return 0
  lax.fori_loop(0, EPT // L, hist_pass, 0)

  pltpu.sync_copy(hist, cnt_sp.at[idn_v], add=True)
  plsc.subcore_barrier()

  lo = sid * (NPT // F)
  floor8 = pl.multiple_of(jnp.minimum((lo // 8) * 8, HR - 16), 8)
  skew = lo - floor8
  pltpu.sync_copy(cnt_sp.at[pl.ds(floor8, 16)],
                  rows_v.at[1, pl.ds(0, 16)])

  def cnt_inv(i, _):
    row = skew + (i // 8)
    col = (i % 8) * L
    v = rows_v[1, row, pl.ds(col, L)]
    rows_v[1, 16 + i // 8, pl.ds(col, L)] = 1.0 / jnp.maximum(v, 1.0)
    return 0
  lax.fori_loop(0, NPT // L, cnt_inv, 0)

  for q in range(NSUB):
    base = sid * NPT + q * RSUB

    @pl.when(base < N)
    def _(q=q, base=base):
      pltpu.sync_copy(acc_sp.at[pl.ds(base, RSUB)], rows_v.at[0])

      def scale_grp(g, _, q=q):
        n = q * RSUB + g * L
        cvec = rows_v[1, 16 + n // F, pl.ds(n % F, L)]
        for k in range(L):
          inv = cvec[k]
          for j in range(F // L):
            sl = pl.ds(j * L, L)
            rows_v[0, g * L + k, sl] = rows_v[0, g * L + k, sl] * inv
        return 0
      lax.fori_loop(0, RSUB // L, scale_grp, 0)

      pltpu.sync_copy(rows_v.at[0], out_hbm.at[pl.ds(cid * N + base, RSUB)])


@jax.jit
def _graph_layer(x2, idx_all):
  mesh = plsc.VectorSubcoreMesh(core_axis_name="c", subcore_axis_name="s")
  return pl.kernel(
      _body,
      out_type=jax.ShapeDtypeStruct((B * N, F), jnp.float32),
      mesh=mesh,
      compiler_params=pltpu.CompilerParams(needs_layout_passes=False),
      scratch_types=[
          pltpu.VMEM_SHARED((N_PAD, F), jnp.float32),
          pltpu.VMEM_SHARED((HR, F), jnp.float32),
          pltpu.VMEM((EPT,), jnp.int32),
          pltpu.VMEM((3, 2, K), jnp.int32),
          pltpu.VMEM((3, K, F), jnp.float32),
          pltpu.VMEM((HR,), jnp.int32),
          pltpu.SemaphoreType.DMA,
          pltpu.SemaphoreType.DMA,
          pltpu.SemaphoreType.DMA,
          pltpu.SemaphoreType.DMA,
          pltpu.SemaphoreType.DMA,
          pltpu.SemaphoreType.DMA,
          pltpu.SemaphoreType.DMA,
      ],
  )(x2, idx_all)


def kernel(X, edge_index):
  x2 = X.reshape(B * N, F)
  src = edge_index[0]
  dst = edge_index[1]
  packed = jnp.stack([src | (dst << 16), (src + N) | (dst << 16)])
  idx_all = packed.reshape(-1)
  out2 = _graph_layer(x2, idx_all)
  return out2.reshape(B, N, F)
